# baseline (speedup 1.0000x reference)
import jax
import jax.numpy as jnp
from jax import lax
from jax.experimental import pallas as pl
from jax.experimental.pallas import tpu as pltpu
from jax.experimental.pallas import tpu_sc as plsc

BATCH = 16384
INPUT_DIM = 39
N_DENSE = 13
N_CAT = 26
EMB_DIM = 16
OUT_DIM = N_DENSE + N_CAT * EMB_DIM

NUM_CORES = 2
NUM_SUBCORES = 16
LANES = 16
NW = NUM_CORES * NUM_SUBCORES
BPW = BATCH // NW


def _encoder_body(xT, *refs):
    tables = refs[:N_CAT]
    outT = refs[N_CAT]
    idx2, dslab, dbuf, gbuf, gbuf2, tbuf, sem, sem2 = refs[N_CAT + 1:]

    wid = lax.axis_index("s") * NUM_CORES + lax.axis_index("c")
    base = pl.multiple_of(wid * jnp.int32(BPW), BPW)
    lane = lax.iota(jnp.int32, LANES)

    pltpu.sync_copy(xT.at[pl.ds(N_DENSE, N_CAT), pl.ds(base, BPW)], idx2)
    pltpu.sync_copy(xT.at[pl.ds(0, N_DENSE), pl.ds(base, BPW)], dslab)

    bufs = (gbuf, gbuf2)
    sems = (sem, sem2)
    copies = [
        pltpu.async_copy(
            tables[i].at[idx2.at[jnp.int32(i)]], bufs[i % 2], sems[i % 2]
        )
        for i in range(2)
    ]

    def dgrp(c, carry):
        c16 = pl.multiple_of(c * jnp.int32(LANES), LANES)
        for j in range(N_DENSE):
            dbuf[jnp.int32(j), pl.ds(c16, LANES)] = dslab[
                jnp.int32(j), pl.ds(c16, LANES)
            ].astype(jnp.float32)
        return carry

    lax.fori_loop(0, BPW // LANES, dgrp, jnp.int32(0))
    pltpu.sync_copy(dbuf, outT.at[pl.ds(0, N_DENSE), pl.ds(base, BPW)])

    for i in range(N_CAT):
        copies[i % 2].wait()
        gb = bufs[i % 2]

        def tgrp(c, carry):
            c16 = pl.multiple_of(c * jnp.int32(LANES), LANES)
            rows = lane + c * jnp.int32(LANES)
            for e in range(EMB_DIM):
                cols = jnp.full((LANES,), e, jnp.int32)
                tbuf[jnp.int32(e), pl.ds(c16, LANES)] = plsc.load_gather(
                    gb, [rows, cols]
                )
            return carry

        lax.fori_loop(0, BPW // LANES, tgrp, jnp.int32(0))
        if i + 2 < N_CAT:
            copies[i % 2] = pltpu.async_copy(
                tables[i + 2].at[idx2.at[jnp.int32(i + 2)]],
                bufs[i % 2],
                sems[i % 2],
            )
        pltpu.sync_copy(
            tbuf,
            outT.at[pl.ds(N_DENSE + i * EMB_DIM, EMB_DIM), pl.ds(base, BPW)],
        )


@jax.jit
def _encode(xT, *tables):
    mesh = plsc.VectorSubcoreMesh(core_axis_name="c", subcore_axis_name="s")
    return pl.kernel(
        _encoder_body,
        mesh=mesh,
        out_type=jax.ShapeDtypeStruct((OUT_DIM, BATCH), jnp.float32),
        scratch_types=[
            pltpu.VMEM((N_CAT, BPW), jnp.int32),
            pltpu.VMEM((N_DENSE, BPW), jnp.int32),
            pltpu.VMEM((N_DENSE, BPW), jnp.float32),
            pltpu.VMEM((BPW, EMB_DIM), jnp.float32),
            pltpu.VMEM((BPW, EMB_DIM), jnp.float32),
            pltpu.VMEM((EMB_DIM, BPW), jnp.float32),
            pltpu.SemaphoreType.DMA,
            pltpu.SemaphoreType.DMA,
        ],
        compiler_params=pltpu.CompilerParams(
            use_tc_tiling_on_sc=False, needs_layout_passes=False
        ),
    )(xT, *tables)


def kernel(x, emb_0, emb_1, emb_2, emb_3, emb_4, emb_5, emb_6, emb_7, emb_8,
           emb_9, emb_10, emb_11, emb_12, emb_13, emb_14, emb_15, emb_16,
           emb_17, emb_18, emb_19, emb_20, emb_21, emb_22, emb_23, emb_24,
           emb_25):
    with jax.enable_x64(False):
        xT = jnp.asarray(x, jnp.int32).T
        outT = _encode(xT, emb_0, emb_1, emb_2, emb_3, emb_4, emb_5, emb_6,
                       emb_7, emb_8, emb_9, emb_10, emb_11, emb_12, emb_13,
                       emb_14, emb_15, emb_16, emb_17, emb_18, emb_19,
                       emb_20, emb_21, emb_22, emb_23, emb_24, emb_25)
        return outT.T


# speedup vs baseline: 1.0138x; 1.0138x over previous
import jax
import jax.numpy as jnp
from jax import lax
from jax.experimental import pallas as pl
from jax.experimental.pallas import tpu as pltpu
from jax.experimental.pallas import tpu_sc as plsc

BATCH = 16384
INPUT_DIM = 39
N_DENSE = 13
N_CAT = 26
EMB_DIM = 16
OUT_DIM = N_DENSE + N_CAT * EMB_DIM
PAD = 3
PADDED = PAD + OUT_DIM

NUM_CORES = 2
NUM_SUBCORES = 16
LANES = 16
NW = NUM_CORES * NUM_SUBCORES
BPW = BATCH // NW


def _encoder_body(xT, *refs):
    tables = refs[:N_CAT]
    out = refs[N_CAT]
    idx2, dslab, dbuf, gbuf, gbuf2, sem, sem2 = refs[N_CAT + 1:]
    wid = lax.axis_index("s") * NUM_CORES + lax.axis_index("c")
    base = pl.multiple_of(wid * jnp.int32(BPW), BPW)
    lane = lax.iota(jnp.int32, LANES)

    pltpu.sync_copy(xT.at[pl.ds(N_DENSE, N_CAT), pl.ds(base, BPW)], idx2)
    pltpu.sync_copy(xT.at[pl.ds(0, N_DENSE), pl.ds(base, BPW)], dslab)

    bufs = (gbuf, gbuf2)
    sems = (sem, sem2)
    copies = [
        pltpu.async_copy(
            tables[i].at[idx2.at[jnp.int32(i)]], bufs[i % 2], sems[i % 2]
        )
        for i in range(2)
    ]

    rowsel = jnp.maximum(lane - jnp.int32(PAD), 0)

    def grp(c, carry):
        r0 = c * jnp.int32(LANES)
        for off in range(LANES):
            r = r0 + jnp.int32(off)
            vals = plsc.load_gather(
                dslab, [rowsel, jnp.full((LANES,), 0, jnp.int32) + r]
            ).astype(jnp.float32)
            dbuf[r, :] = vals
        return carry

    lax.fori_loop(0, BPW // LANES, grp, jnp.int32(0))
    pltpu.sync_copy(dbuf, out.at[pl.ds(base, BPW), pl.ds(0, PAD + N_DENSE)])

    for i in range(N_CAT):
        copies[i % 2].wait()
        pltpu.sync_copy(
            bufs[i % 2],
            out.at[pl.ds(base, BPW), pl.ds(PAD + N_DENSE + i * EMB_DIM, EMB_DIM)],
        )
        if i + 2 < N_CAT:
            copies[i % 2] = pltpu.async_copy(
                tables[i + 2].at[idx2.at[jnp.int32(i + 2)]],
                bufs[i % 2],
                sems[i % 2],
            )


@jax.jit
def _encode(xT, *tables):
    mesh = plsc.VectorSubcoreMesh(core_axis_name="c", subcore_axis_name="s")
    padded = pl.kernel(
        _encoder_body,
        mesh=mesh,
        out_type=jax.ShapeDtypeStruct((BATCH, PADDED), jnp.float32),
        scratch_types=[
            pltpu.VMEM((N_CAT, BPW), jnp.int32),
            pltpu.VMEM((N_DENSE, BPW), jnp.int32),
            pltpu.VMEM((BPW, PAD + N_DENSE), jnp.float32),
            pltpu.VMEM((BPW, EMB_DIM), jnp.float32),
            pltpu.VMEM((BPW, EMB_DIM), jnp.float32),
            pltpu.SemaphoreType.DMA,
            pltpu.SemaphoreType.DMA,
        ],
        compiler_params=pltpu.CompilerParams(
            use_tc_tiling_on_sc=False, needs_layout_passes=False
        ),
    )(xT, *tables)
    return padded[:, PAD:]


def kernel(x, emb_0, emb_1, emb_2, emb_3, emb_4, emb_5, emb_6, emb_7, emb_8,
           emb_9, emb_10, emb_11, emb_12, emb_13, emb_14, emb_15, emb_16,
           emb_17, emb_18, emb_19, emb_20, emb_21, emb_22, emb_23, emb_24,
           emb_25):
    with jax.enable_x64(False):
        xT = jnp.asarray(x, jnp.int32).T
        return _encode(xT, emb_0, emb_1, emb_2, emb_3, emb_4, emb_5, emb_6,
                       emb_7, emb_8, emb_9, emb_10, emb_11, emb_12, emb_13,
                       emb_14, emb_15, emb_16, emb_17, emb_18, emb_19,
                       emb_20, emb_21, emb_22, emb_23, emb_24, emb_25)
